# baseline (speedup 1.0000x reference)
import functools
import jax
import jax.numpy as jnp
from jax import lax
from jax.experimental import pallas as pl
from jax.experimental.pallas import tpu as pltpu
from jax.experimental.pallas import tpu_sc as plsc

HEAD_DIM = 64
NUM_EMB = 257
K_PAD = 384
SEQ = 2048
NC = 2
NS = 16
NW = NC * NS
NI_PER_W = SEQ // NW
JCHUNK = 512
NJC = SEQ // JCHUNK
NG = JCHUNK // 16
N_PAIRS = NI_PER_W * NJC // 2


def _sc_body(idx_hbm, table_hbm, out_hbm, table_v, idx_v, m_v, sems, isem):
    wid = lax.axis_index("s") * NC + lax.axis_index("c")
    i0 = wid * NI_PER_W
    pltpu.sync_copy(table_hbm, table_v)
    pltpu.sync_copy(idx_hbm.at[i0], idx_v.at[0])

    def make_chunk(half):
        def chunk(k):
            t = k // NJC
            i = i0 + t
            tb = lax.rem(t, 2)
            jc = lax.rem(k, NJC)
            jof = pl.multiple_of(jc * JCHUNK, JCHUNK)

            @pl.when(jnp.logical_and(jc == 0, t > 0))
            def _():
                pltpu.make_async_copy(
                    idx_hbm.at[i0], idx_v.at[0], isem).wait()

            @pl.when(jnp.logical_and(jc == 0, t < NI_PER_W - 1))
            def _():
                pltpu.async_copy(
                    idx_hbm.at[i + 1], idx_v.at[1 - tb], isem)

            @plsc.parallel_loop(0, NG, unroll=8)
            def g_body(g):
                gg = jc * NG + g
                v = idx_v[tb, gg // 8, pl.ds(lax.rem(gg, 8) * 16, 16)]
                for d in range(HEAD_DIM):
                    row = table_v.at[pl.ds(d * K_PAD, K_PAD)]
                    vals = plsc.load_gather(row, [v])
                    m_v[half, d, pl.ds(g * 16, 16)] = vals
            pltpu.async_copy(
                m_v.at[half],
                out_hbm.at[i, :, pl.ds(jof, JCHUNK)],
                sems.at[half])
        return chunk

    chunk0 = make_chunk(0)
    chunk1 = make_chunk(1)

    def pair_body(p, carry):
        @pl.when(p > 0)
        def _():
            pltpu.make_async_copy(
                m_v.at[0], out_hbm.at[i0, :, pl.ds(0, JCHUNK)],
                sems.at[0]).wait()
        chunk0(2 * p)

        @pl.when(p > 0)
        def _():
            pltpu.make_async_copy(
                m_v.at[1], out_hbm.at[i0, :, pl.ds(0, JCHUNK)],
                sems.at[1]).wait()
        chunk1(2 * p + 1)
        return carry

    lax.fori_loop(0, N_PAIRS, pair_body, 0)
    pltpu.make_async_copy(
        m_v.at[0], out_hbm.at[i0, :, pl.ds(0, JCHUNK)], sems.at[0]).wait()
    pltpu.make_async_copy(
        m_v.at[1], out_hbm.at[i0, :, pl.ds(0, JCHUNK)], sems.at[1]).wait()


_sc_gather = functools.partial(
    pl.kernel,
    out_type=jax.ShapeDtypeStruct((SEQ, HEAD_DIM, SEQ), jnp.float32),
    mesh=plsc.VectorSubcoreMesh(core_axis_name="c", subcore_axis_name="s"),
    scratch_types=[
        pltpu.VMEM((HEAD_DIM * K_PAD,), jnp.float32),
        pltpu.VMEM((2, 16, 128), jnp.int32),
        pltpu.VMEM((2, HEAD_DIM, JCHUNK), jnp.float32),
        pltpu.SemaphoreType.DMA((2,)),
        pltpu.SemaphoreType.DMA,
    ],
    compiler_params=pltpu.CompilerParams(needs_layout_passes=False),
)(_sc_body)


def kernel(input, embeddings):
    idx3 = input.reshape(SEQ, 16, 128).astype(jnp.int32)
    table_t = jnp.zeros((HEAD_DIM, K_PAD), jnp.float32)
    table_t = table_t.at[:, :NUM_EMB].set(embeddings.T)
    out3 = _sc_gather(idx3, table_t.reshape(-1))
    return jnp.transpose(out3, (0, 2, 1))


# speedup vs baseline: 1.0867x; 1.0867x over previous
import functools
import jax
import jax.numpy as jnp
from jax import lax
from jax.experimental import pallas as pl
from jax.experimental.pallas import tpu as pltpu
from jax.experimental.pallas import tpu_sc as plsc

HEAD_DIM = 64
NUM_EMB = 257
K_PAD = 384
SEQ = 2048
NC = 2
NS = 16
NW = NC * NS
NI_PER_W = SEQ // NW
JCHUNK = 512
NJC = SEQ // JCHUNK
NG = JCHUNK // 16
N_PAIRS = NI_PER_W * NJC // 2


def _sc_body(idx_hbm, table_hbm, out_hbm, table_v, idx_v, m_v, sems, isem):
    wid = lax.axis_index("s") * NC + lax.axis_index("c")
    i0 = wid * NI_PER_W
    pltpu.sync_copy(table_hbm, table_v)
    pltpu.sync_copy(idx_hbm.at[i0], idx_v.at[0])

    def make_chunk(half):
        def chunk(k):
            t = k // NJC
            i = i0 + t
            tb = lax.rem(t, 2)
            jc = lax.rem(k, NJC)
            jof = pl.multiple_of(jc * JCHUNK, JCHUNK)

            @pl.when(jnp.logical_and(jc == 0, t > 0))
            def _():
                pltpu.make_async_copy(
                    idx_hbm.at[i0], idx_v.at[0], isem).wait()

            @pl.when(jnp.logical_and(jc == 0, t < NI_PER_W - 1))
            def _():
                pltpu.async_copy(
                    idx_hbm.at[i + 1], idx_v.at[1 - tb], isem)

            @plsc.parallel_loop(0, NG, unroll=4)
            def g_body(g):
                gg = jc * NG + g
                v = idx_v[tb, gg // 8, pl.ds(lax.rem(gg, 8) * 16, 16)]
                for d in range(HEAD_DIM):
                    row = table_v.at[pl.ds(d * K_PAD, K_PAD)]
                    vals = plsc.load_gather(row, [v])
                    m_v[half, d, pl.ds(g * 16, 16)] = vals
            pltpu.async_copy(
                m_v.at[half],
                out_hbm.at[i, :, pl.ds(jof, JCHUNK)],
                sems.at[half])
        return chunk

    chunk0 = make_chunk(0)
    chunk1 = make_chunk(1)

    def pair_body(p, carry):
        @pl.when(p > 0)
        def _():
            pltpu.make_async_copy(
                m_v.at[0], out_hbm.at[i0, :, pl.ds(0, JCHUNK)],
                sems.at[0]).wait()
        chunk0(2 * p)

        @pl.when(p > 0)
        def _():
            pltpu.make_async_copy(
                m_v.at[1], out_hbm.at[i0, :, pl.ds(0, JCHUNK)],
                sems.at[1]).wait()
        chunk1(2 * p + 1)
        return carry

    lax.fori_loop(0, N_PAIRS, pair_body, 0)
    pltpu.make_async_copy(
        m_v.at[0], out_hbm.at[i0, :, pl.ds(0, JCHUNK)], sems.at[0]).wait()
    pltpu.make_async_copy(
        m_v.at[1], out_hbm.at[i0, :, pl.ds(0, JCHUNK)], sems.at[1]).wait()


_sc_gather = functools.partial(
    pl.kernel,
    out_type=jax.ShapeDtypeStruct((SEQ, HEAD_DIM, SEQ), jnp.float32),
    mesh=plsc.VectorSubcoreMesh(core_axis_name="c", subcore_axis_name="s"),
    scratch_types=[
        pltpu.VMEM((HEAD_DIM * K_PAD,), jnp.float32),
        pltpu.VMEM((2, 16, 128), jnp.int32),
        pltpu.VMEM((2, HEAD_DIM, JCHUNK), jnp.float32),
        pltpu.SemaphoreType.DMA((2,)),
        pltpu.SemaphoreType.DMA,
    ],
    compiler_params=pltpu.CompilerParams(needs_layout_passes=False),
)(_sc_body)


def kernel(input, embeddings):
    idx3 = input.reshape(SEQ, 16, 128).astype(jnp.int32)
    table_t = jnp.zeros((HEAD_DIM, K_PAD), jnp.float32)
    table_t = table_t.at[:, :NUM_EMB].set(embeddings.T)
    out3 = _sc_gather(idx3, table_t.reshape(-1))
    return jnp.transpose(out3, (0, 2, 1))
